# Initial kernel scaffold; baseline (speedup 1.0000x reference)
import jax
import jax.numpy as jnp
from jax.experimental import pallas as pl


def kernel(x, edge_index, edge_weight, edges, W):
    raise NotImplementedError("write your pallas kernel here")


# speedup vs baseline: 2.8457x; 2.8457x over previous
import functools

import jax
import jax.numpy as jnp
from jax import lax
from jax.experimental import pallas as pl
from jax.experimental.pallas import tpu as pltpu
from jax.experimental.pallas import tpu_sc as plsc

_NTILES = 32
_CE = 128
_CQ = 64


def _segment_sum_partials(x, src, dst, ew):
    n, d = x.shape
    e = src.shape[0]
    nchunks = e // _CE
    iters = -(-nchunks // _NTILES)
    zrows = 40
    ngroups = n // zrows
    mesh = plsc.VectorSubcoreMesh(core_axis_name="c", subcore_axis_name="s")

    @functools.partial(
        pl.kernel,
        mesh=mesh,
        out_type=jax.ShapeDtypeStruct((2, n, d), jnp.float32),
        scratch_types=[
            pltpu.VMEM((_CE,), jnp.int32),
            pltpu.VMEM((_CE,), jnp.int32),
            pltpu.VMEM((_CE,), jnp.float32),
            pltpu.VMEM((_CE, d), jnp.float32),
            pltpu.VMEM((zrows, d), jnp.float32),
            pltpu.VMEM_SHARED((n, d), jnp.float32),
            pltpu.SemaphoreType.DMA,
        ],
    )
    def k(x_hbm, src_hbm, dst_hbm, ew_hbm, part_hbm,
          sidx_v, didx_v, w_v, rows_v, zero_v, shared, sem):
        c = lax.axis_index("c")
        s = lax.axis_index("s")
        wid = s * 2 + c

        zvec = jnp.zeros((16,), jnp.float32)
        for r in range(zrows):
            for db in range(d // 16):
                zero_v[r, pl.ds(db * 16, 16)] = zvec

        def zero_body(i, carry):
            g = i * 16 + s

            @pl.when(g < ngroups)
            def _():
                pltpu.sync_copy(zero_v, shared.at[pl.ds(g * zrows, zrows)])

            return carry

        lax.fori_loop(0, -(-ngroups // 16), zero_body, 0)
        plsc.subcore_barrier()

        def chunk_body(kk, carry):
            j = kk * _NTILES + wid

            @pl.when(j < nchunks)
            def _():
                base = j * _CE
                pltpu.sync_copy(src_hbm.at[pl.ds(base, _CE)], sidx_v)
                pltpu.sync_copy(dst_hbm.at[pl.ds(base, _CE)], didx_v)
                pltpu.sync_copy(ew_hbm.at[pl.ds(base, _CE)], w_v)
                pltpu.async_copy(x_hbm.at[sidx_v], rows_v, sem).wait()

                def scale_body(g, carry2):
                    w16 = w_v[pl.ds(g * 16, 16)]
                    for l in range(16):
                        w = w16[l]
                        ei = g * 16 + l
                        for db in range(d // 16):
                            sl = pl.ds(db * 16, 16)
                            rows_v[ei, sl] = rows_v[ei, sl] * w
                    return carry2

                lax.fori_loop(0, _CE // 16, scale_body, 0)
                pltpu.sync_copy(rows_v, shared.at[didx_v], add=True)

            return carry

        lax.fori_loop(0, iters, chunk_body, 0)
        plsc.subcore_barrier()

        def out_body(i, carry):
            g = i * 16 + s

            @pl.when(g < ngroups)
            def _():
                pltpu.sync_copy(shared.at[pl.ds(g * zrows, zrows)],
                                part_hbm.at[c, pl.ds(g * zrows, zrows)])

            return carry

        lax.fori_loop(0, -(-ngroups // 16), out_body, 0)

    return k(x, src, dst, ew)


def _linear(part0, part1, w):
    n, d = part0.shape
    blk = 400

    def mm(a_ref, b_ref, w_ref, o_ref):
        o_ref[...] = jnp.dot(a_ref[...] + b_ref[...], w_ref[...],
                             preferred_element_type=jnp.float32)

    return pl.pallas_call(
        mm,
        grid=(n // blk,),
        in_specs=[
            pl.BlockSpec((blk, d), lambda i: (i, 0)),
            pl.BlockSpec((blk, d), lambda i: (i, 0)),
            pl.BlockSpec((d, d), lambda i: (0, 0)),
        ],
        out_specs=pl.BlockSpec((blk, d), lambda i: (i, 0)),
        out_shape=jax.ShapeDtypeStruct((n, d), jnp.float32),
    )(part0, part1, w)


def _gather_pairs(h, e0, e1):
    n, d = h.shape
    q = e0.shape[0]
    nchunks = q // _CQ
    iters = -(-nchunks // _NTILES)
    mesh = plsc.VectorSubcoreMesh(core_axis_name="c", subcore_axis_name="s")

    @functools.partial(
        pl.kernel,
        mesh=mesh,
        out_type=(jax.ShapeDtypeStruct((q, d), jnp.float32),
                  jax.ShapeDtypeStruct((q, d), jnp.float32)),
        scratch_types=[
            pltpu.VMEM((_CQ,), jnp.int32),
            pltpu.VMEM((_CQ, d), jnp.float32),
            pltpu.SemaphoreType.DMA,
        ],
    )
    def k(h_hbm, e0_hbm, e1_hbm, ha_hbm, hb_hbm, idx_v, rows_v, sem):
        c = lax.axis_index("c")
        s = lax.axis_index("s")
        wid = s * 2 + c

        def chunk_body(kk, carry):
            j = kk * _NTILES + wid

            @pl.when(j < nchunks)
            def _():
                base = j * _CQ
                pltpu.sync_copy(e0_hbm.at[pl.ds(base, _CQ)], idx_v)
                pltpu.async_copy(h_hbm.at[idx_v], rows_v, sem).wait()
                pltpu.sync_copy(rows_v, ha_hbm.at[pl.ds(base, _CQ)])
                pltpu.sync_copy(e1_hbm.at[pl.ds(base, _CQ)], idx_v)
                pltpu.async_copy(h_hbm.at[idx_v], rows_v, sem).wait()
                pltpu.sync_copy(rows_v, hb_hbm.at[pl.ds(base, _CQ)])

            return carry

        lax.fori_loop(0, iters, chunk_body, 0)

    return k(h, e0, e1)


def _pair_dot(ha, hb):
    q, d = ha.shape
    blk = 1000

    def dot_body(a_ref, b_ref, o_ref):
        o_ref[...] = jnp.sum(a_ref[...] * b_ref[...], axis=1, keepdims=True)

    out = pl.pallas_call(
        dot_body,
        grid=(q // blk,),
        in_specs=[
            pl.BlockSpec((blk, d), lambda i: (i, 0)),
            pl.BlockSpec((blk, d), lambda i: (i, 0)),
        ],
        out_specs=pl.BlockSpec((blk, 1), lambda i: (i, 0)),
        out_shape=jax.ShapeDtypeStruct((q, 1), jnp.float32),
    )(ha, hb)
    return out.reshape(q)


def kernel(x, edge_index, edge_weight, edges, W):
    src = edge_index[0]
    dst = edge_index[1]
    part = _segment_sum_partials(x, src, dst, edge_weight)
    h = _linear(part[0], part[1], W)
    ha, hb = _gather_pairs(h, edges[0], edges[1])
    return _pair_dot(ha, hb)


# speedup vs baseline: 5.0348x; 1.7693x over previous
import functools

import jax
import jax.numpy as jnp
from jax import lax
from jax.experimental import pallas as pl
from jax.experimental.pallas import tpu as pltpu
from jax.experimental.pallas import tpu_sc as plsc

_NTILES = 32
_CE = 128
_CQ = 64


def _segment_sum_partials(x, epack, ew):
    n, d = x.shape
    nchunks = epack.shape[0]
    iters = -(-nchunks // _NTILES)
    zrows = 40
    ngroups = n // zrows
    mesh = plsc.VectorSubcoreMesh(core_axis_name="c", subcore_axis_name="s")

    @functools.partial(
        pl.kernel,
        mesh=mesh,
        out_type=jax.ShapeDtypeStruct((2, n, d), jnp.float32),
        scratch_types=[
            pltpu.VMEM((2, 2, _CE), jnp.int32),
            pltpu.VMEM((2, _CE), jnp.float32),
            pltpu.VMEM((2, _CE, d), jnp.float32),
            pltpu.VMEM((zrows, d), jnp.float32),
            pltpu.VMEM_SHARED((n, d), jnp.float32),
            pltpu.SemaphoreType.DMA,
            pltpu.SemaphoreType.DMA,
            pltpu.SemaphoreType.DMA,
        ],
    )
    def k(x_hbm, epack_hbm, ew_hbm, part_hbm, idxw_v, w_v, rows_v, zero_v,
          shared, sem_i, sem_w, sem_g):
        c = lax.axis_index("c")
        s = lax.axis_index("s")
        wid = s * 2 + c

        zvec = jnp.zeros((16,), jnp.float32)
        for r in range(zrows):
            for db in range(d // 16):
                zero_v[r, pl.ds(db * 16, 16)] = zvec

        def zero_body(i, carry):
            g = i * 16 + s

            @pl.when(g < ngroups)
            def _():
                pltpu.sync_copy(zero_v, shared.at[pl.ds(g * zrows, zrows)])

            return carry

        lax.fori_loop(0, -(-ngroups // 16), zero_body, 0)
        plsc.subcore_barrier()

        pltpu.sync_copy(epack_hbm.at[wid], idxw_v.at[0])
        pltpu.sync_copy(ew_hbm.at[wid], w_v.at[0])
        pltpu.async_copy(x_hbm.at[idxw_v.at[0, 0]], rows_v.at[0], sem_g)

        def scale_rows(b):
            def scale_body(g, carry2):
                w16 = w_v[b, pl.ds(g * 16, 16)]
                for l in range(16):
                    w = w16[l]
                    ei = g * 16 + l
                    for db in range(d // 16):
                        sl = pl.ds(db * 16, 16)
                        rows_v[b, ei, sl] = rows_v[b, ei, sl] * w
                return carry2

            lax.fori_loop(0, _CE // 16, scale_body, 0)

        def outer(i, carry):
            for b in range(2):
                nb = 1 - b
                kk = i * 2 + b
                j = kk * _NTILES + wid
                jn = j + _NTILES

                @pl.when(jn < nchunks)
                def _():
                    pltpu.async_copy(epack_hbm.at[jn], idxw_v.at[nb], sem_i)
                    pltpu.async_copy(ew_hbm.at[jn], w_v.at[nb], sem_w)

                @pl.when(j < nchunks)
                def _():
                    pltpu.make_async_copy(
                        x_hbm.at[idxw_v.at[b, 0]], rows_v.at[b], sem_g).wait()

                @pl.when(jn < nchunks)
                def _():
                    pltpu.make_async_copy(
                        epack_hbm.at[0], idxw_v.at[nb], sem_i).wait()
                    pltpu.make_async_copy(
                        ew_hbm.at[0], w_v.at[nb], sem_w).wait()
                    pltpu.async_copy(
                        x_hbm.at[idxw_v.at[nb, 0]], rows_v.at[nb], sem_g)

                @pl.when(j < nchunks)
                def _():
                    scale_rows(b)
                    pltpu.sync_copy(rows_v.at[b], shared.at[idxw_v.at[b, 1]],
                                    add=True)

            return carry

        lax.fori_loop(0, iters // 2 + 1, outer, 0)
        plsc.subcore_barrier()

        def out_body(i, carry):
            g = i * 16 + s

            @pl.when(g < ngroups)
            def _():
                pltpu.sync_copy(shared.at[pl.ds(g * zrows, zrows)],
                                part_hbm.at[c, pl.ds(g * zrows, zrows)])

            return carry

        lax.fori_loop(0, -(-ngroups // 16), out_body, 0)

    return k(x, epack, ew)


def _linear(part0, part1, w):
    n, d = part0.shape
    blk = 400

    def mm(a_ref, b_ref, w_ref, o_ref):
        o_ref[...] = jnp.dot(a_ref[...] + b_ref[...], w_ref[...],
                             preferred_element_type=jnp.float32)

    return pl.pallas_call(
        mm,
        grid=(n // blk,),
        in_specs=[
            pl.BlockSpec((blk, d), lambda i: (i, 0)),
            pl.BlockSpec((blk, d), lambda i: (i, 0)),
            pl.BlockSpec((d, d), lambda i: (0, 0)),
        ],
        out_specs=pl.BlockSpec((blk, d), lambda i: (i, 0)),
        out_shape=jax.ShapeDtypeStruct((n, d), jnp.float32),
    )(part0, part1, w)


def _pair_partial_dots(h, qpack):
    n, d = h.shape
    nchunks = qpack.shape[0]
    iters = -(-nchunks // _NTILES)
    mesh = plsc.VectorSubcoreMesh(core_axis_name="c", subcore_axis_name="s")

    @functools.partial(
        pl.kernel,
        mesh=mesh,
        out_type=jax.ShapeDtypeStruct((nchunks, _CQ, 16), jnp.float32),
        scratch_types=[
            pltpu.VMEM((2, 2, _CQ), jnp.int32),
            pltpu.VMEM((2, 2, _CQ, d), jnp.float32),
            pltpu.VMEM((2, _CQ, 16), jnp.float32),
            pltpu.SemaphoreType.DMA,
            pltpu.SemaphoreType.DMA,
            pltpu.SemaphoreType.DMA,
            pltpu.SemaphoreType.DMA,
        ],
    )
    def k(h_hbm, qpack_hbm, out_hbm, pairs_v, rows_v, pbuf_v,
          sem_p, sem_a, sem_b, sem_o):
        c = lax.axis_index("c")
        s = lax.axis_index("s")
        wid = s * 2 + c

        pltpu.sync_copy(qpack_hbm.at[wid], pairs_v.at[0])
        pltpu.async_copy(h_hbm.at[pairs_v.at[0, 0]], rows_v.at[0, 0], sem_a)
        pltpu.async_copy(h_hbm.at[pairs_v.at[0, 1]], rows_v.at[0, 1], sem_b)

        def compute_chunk(b):
            def dot_body(g, carry2):
                for l in range(16):
                    qi = g * 16 + l
                    acc = None
                    for db in range(d // 16):
                        sl = pl.ds(db * 16, 16)
                        prod = rows_v[b, 0, qi, sl] * rows_v[b, 1, qi, sl]
                        acc = prod if acc is None else acc + prod
                    pbuf_v[b, qi, :] = acc
                return carry2

            lax.fori_loop(0, _CQ // 16, dot_body, 0)

        def outer(i, carry):
            for b in range(2):
                nb = 1 - b
                kk = i * 2 + b
                j = kk * _NTILES + wid
                jn = j + _NTILES

                @pl.when(jn < nchunks)
                def _():
                    pltpu.async_copy(qpack_hbm.at[jn], pairs_v.at[nb], sem_p)

                @pl.when(j < nchunks)
                def _():
                    pltpu.make_async_copy(
                        h_hbm.at[pairs_v.at[b, 0]], rows_v.at[b, 0],
                        sem_a).wait()
                    pltpu.make_async_copy(
                        h_hbm.at[pairs_v.at[b, 1]], rows_v.at[b, 1],
                        sem_b).wait()

                @pl.when(jn < nchunks)
                def _():
                    pltpu.make_async_copy(
                        qpack_hbm.at[0], pairs_v.at[nb], sem_p).wait()
                    pltpu.async_copy(
                        h_hbm.at[pairs_v.at[nb, 0]], rows_v.at[nb, 0], sem_a)
                    pltpu.async_copy(
                        h_hbm.at[pairs_v.at[nb, 1]], rows_v.at[nb, 1], sem_b)

                @pl.when((kk >= 2) & (j < nchunks))
                def _():
                    pltpu.make_async_copy(
                        pbuf_v.at[b], out_hbm.at[0], sem_o).wait()

                @pl.when(j < nchunks)
                def _():
                    compute_chunk(b)
                    pltpu.async_copy(pbuf_v.at[b], out_hbm.at[j], sem_o)

            return carry

        lax.fori_loop(0, iters // 2 + 1, outer, 0)
        pltpu.make_async_copy(pbuf_v.at[0], out_hbm.at[0], sem_o).wait()
        pltpu.make_async_copy(pbuf_v.at[1], out_hbm.at[0], sem_o).wait()

    return k(h, qpack)


def _lane_reduce(partials):
    q = partials.shape[0]
    blk = 8000

    def red(p_ref, o_ref):
        o_ref[...] = jnp.sum(p_ref[...], axis=1, keepdims=True)

    out = pl.pallas_call(
        red,
        grid=(q // blk,),
        in_specs=[pl.BlockSpec((blk, 16), lambda i: (i, 0))],
        out_specs=pl.BlockSpec((blk, 1), lambda i: (i, 0)),
        out_shape=jax.ShapeDtypeStruct((q, 1), jnp.float32),
    )(partials)
    return out.reshape(q)


def kernel(x, edge_index, edge_weight, edges, W):
    e = edge_index.shape[1]
    q = edges.shape[1]
    epack = jnp.stack(
        [edge_index[0].reshape(e // _CE, _CE),
         edge_index[1].reshape(e // _CE, _CE)], axis=1)
    ew = edge_weight.reshape(e // _CE, _CE)
    qpack = jnp.stack(
        [edges[0].reshape(q // _CQ, _CQ),
         edges[1].reshape(q // _CQ, _CQ)], axis=1)

    part = _segment_sum_partials(x, epack, ew)
    h = _linear(part[0], part[1], W)
    partials = _pair_partial_dots(h, qpack)
    return _lane_reduce(partials.reshape(q, 16))


# speedup vs baseline: 7.1050x; 1.4112x over previous
import functools

import jax
import jax.numpy as jnp
from jax import lax
from jax.experimental import pallas as pl
from jax.experimental.pallas import tpu as pltpu
from jax.experimental.pallas import tpu_sc as plsc

_NTILES = 32
_CE = 128
_CQ = 64


def _segment_sum_partials(x, epack, ew):
    n, d = x.shape
    nchunks = epack.shape[0]
    iters = -(-nchunks // _NTILES)
    zrows = 40
    ngroups = n // zrows
    mesh = plsc.VectorSubcoreMesh(core_axis_name="c", subcore_axis_name="s")

    @functools.partial(
        pl.kernel,
        mesh=mesh,
        out_type=jax.ShapeDtypeStruct((2, n, d), jnp.float32),
        scratch_types=[
            pltpu.VMEM((2, 2, _CE), jnp.int32),
            pltpu.VMEM((2, _CE), jnp.float32),
            pltpu.VMEM((2, _CE, d), jnp.float32),
            pltpu.VMEM((zrows, d), jnp.float32),
            pltpu.VMEM_SHARED((n, d), jnp.float32),
            pltpu.SemaphoreType.DMA,
            pltpu.SemaphoreType.DMA,
            pltpu.SemaphoreType.DMA,
        ],
    )
    def k(x_hbm, epack_hbm, ew_hbm, part_hbm, idxw_v, w_v, rows_v, zero_v,
          shared, sem_i, sem_w, sem_g):
        c = lax.axis_index("c")
        s = lax.axis_index("s")
        wid = s * 2 + c

        zvec = jnp.zeros((16,), jnp.float32)
        for r in range(zrows):
            for db in range(d // 16):
                zero_v[r, pl.ds(db * 16, 16)] = zvec

        def zero_body(i, carry):
            g = i * 16 + s

            @pl.when(g < ngroups)
            def _():
                pltpu.sync_copy(zero_v, shared.at[pl.ds(g * zrows, zrows)])

            return carry

        lax.fori_loop(0, -(-ngroups // 16), zero_body, 0)
        plsc.subcore_barrier()

        pltpu.sync_copy(epack_hbm.at[wid], idxw_v.at[0])
        pltpu.sync_copy(ew_hbm.at[wid], w_v.at[0])
        pltpu.async_copy(x_hbm.at[idxw_v.at[0, 0]], rows_v.at[0], sem_g)

        def scale_rows(b):
            def scale_body(g, carry2):
                w16 = w_v[b, pl.ds(g * 16, 16)]
                for l in range(16):
                    w = w16[l]
                    ei = g * 16 + l
                    for db in range(d // 16):
                        sl = pl.ds(db * 16, 16)
                        rows_v[b, ei, sl] = rows_v[b, ei, sl] * w
                return carry2

            lax.fori_loop(0, _CE // 16, scale_body, 0)

        def outer(i, carry):
            for b in range(2):
                nb = 1 - b
                kk = i * 2 + b
                j = kk * _NTILES + wid
                jn = j + _NTILES

                @pl.when(jn < nchunks)
                def _():
                    pltpu.async_copy(epack_hbm.at[jn], idxw_v.at[nb], sem_i)
                    pltpu.async_copy(ew_hbm.at[jn], w_v.at[nb], sem_w)

                @pl.when(j < nchunks)
                def _():
                    pltpu.make_async_copy(
                        x_hbm.at[idxw_v.at[b, 0]], rows_v.at[b], sem_g).wait()

                @pl.when(jn < nchunks)
                def _():
                    pltpu.make_async_copy(
                        epack_hbm.at[0], idxw_v.at[nb], sem_i).wait()
                    pltpu.make_async_copy(
                        ew_hbm.at[0], w_v.at[nb], sem_w).wait()
                    pltpu.async_copy(
                        x_hbm.at[idxw_v.at[nb, 0]], rows_v.at[nb], sem_g)

                @pl.when(j < nchunks)
                def _():
                    scale_rows(b)
                    pltpu.sync_copy(rows_v.at[b], shared.at[idxw_v.at[b, 1]],
                                    add=True)

            return carry

        lax.fori_loop(0, iters // 2 + 1, outer, 0)
        plsc.subcore_barrier()

        def out_body(i, carry):
            g = i * 16 + s

            @pl.when(g < ngroups)
            def _():
                pltpu.sync_copy(shared.at[pl.ds(g * zrows, zrows)],
                                part_hbm.at[c, pl.ds(g * zrows, zrows)])

            return carry

        lax.fori_loop(0, -(-ngroups // 16), out_body, 0)

    return k(x, epack, ew)


def _linear(part0, part1, w):
    n, d = part0.shape
    blk = 400

    def mm(a_ref, b_ref, w_ref, o_ref):
        o_ref[...] = jnp.dot(a_ref[...] + b_ref[...], w_ref[...],
                             preferred_element_type=jnp.float32)

    return pl.pallas_call(
        mm,
        grid=(n // blk,),
        in_specs=[
            pl.BlockSpec((blk, d), lambda i: (i, 0)),
            pl.BlockSpec((blk, d), lambda i: (i, 0)),
            pl.BlockSpec((d, d), lambda i: (0, 0)),
        ],
        out_specs=pl.BlockSpec((blk, d), lambda i: (i, 0)),
        out_shape=jax.ShapeDtypeStruct((n, d), jnp.float32),
    )(part0, part1, w)


def _pair_partial_dots(h, qpack):
    n, d = h.shape
    nchunks = qpack.shape[0]
    iters = -(-nchunks // _NTILES)
    mesh = plsc.VectorSubcoreMesh(core_axis_name="c", subcore_axis_name="s")

    @functools.partial(
        pl.kernel,
        mesh=mesh,
        out_type=jax.ShapeDtypeStruct((nchunks * 8, 128), jnp.float32),
        scratch_types=[
            pltpu.VMEM((2, 2, _CQ), jnp.int32),
            pltpu.VMEM((2, 2, _CQ, d), jnp.float32),
            pltpu.VMEM((2, 8, 128), jnp.float32),
            pltpu.SemaphoreType.DMA,
            pltpu.SemaphoreType.DMA,
            pltpu.SemaphoreType.DMA,
            pltpu.SemaphoreType.DMA,
        ],
    )
    def k(h_hbm, qpack_hbm, out_hbm, pairs_v, rows_v, pbuf_v,
          sem_p, sem_a, sem_b, sem_o):
        c = lax.axis_index("c")
        s = lax.axis_index("s")
        wid = s * 2 + c

        pltpu.sync_copy(qpack_hbm.at[wid], pairs_v.at[0])
        pltpu.async_copy(h_hbm.at[pairs_v.at[0, 0]], rows_v.at[0, 0], sem_a)
        pltpu.async_copy(h_hbm.at[pairs_v.at[0, 1]], rows_v.at[0, 1], sem_b)

        def compute_chunk(b):
            def dot_body(g, carry2):
                for l in range(16):
                    qi = g * 16 + l
                    acc = None
                    for db in range(d // 16):
                        sl = pl.ds(db * 16, 16)
                        prod = rows_v[b, 0, qi, sl] * rows_v[b, 1, qi, sl]
                        acc = prod if acc is None else acc + prod
                    pbuf_v[b, g * 2 + l // 8, pl.ds((l % 8) * 16, 16)] = acc
                return carry2

            lax.fori_loop(0, _CQ // 16, dot_body, 0)

        def outer(i, carry):
            for b in range(2):
                nb = 1 - b
                kk = i * 2 + b
                j = kk * _NTILES + wid
                jn = j + _NTILES

                @pl.when(jn < nchunks)
                def _():
                    pltpu.async_copy(qpack_hbm.at[jn], pairs_v.at[nb], sem_p)

                @pl.when(j < nchunks)
                def _():
                    pltpu.make_async_copy(
                        h_hbm.at[pairs_v.at[b, 0]], rows_v.at[b, 0],
                        sem_a).wait()
                    pltpu.make_async_copy(
                        h_hbm.at[pairs_v.at[b, 1]], rows_v.at[b, 1],
                        sem_b).wait()

                @pl.when(jn < nchunks)
                def _():
                    pltpu.make_async_copy(
                        qpack_hbm.at[0], pairs_v.at[nb], sem_p).wait()
                    pltpu.async_copy(
                        h_hbm.at[pairs_v.at[nb, 0]], rows_v.at[nb, 0], sem_a)
                    pltpu.async_copy(
                        h_hbm.at[pairs_v.at[nb, 1]], rows_v.at[nb, 1], sem_b)

                @pl.when((kk >= 2) & (j < nchunks))
                def _():
                    pltpu.make_async_copy(
                        pbuf_v.at[b], out_hbm.at[pl.ds(0, 8)], sem_o).wait()

                @pl.when(j < nchunks)
                def _():
                    compute_chunk(b)
                    pltpu.async_copy(
                        pbuf_v.at[b], out_hbm.at[pl.ds(j * 8, 8)], sem_o)

            return carry

        lax.fori_loop(0, iters // 2 + 1, outer, 0)
        pltpu.make_async_copy(pbuf_v.at[0], out_hbm.at[pl.ds(0, 8)], sem_o).wait()
        pltpu.make_async_copy(pbuf_v.at[1], out_hbm.at[pl.ds(0, 8)], sem_o).wait()

    return k(h, qpack)


def _lane_reduce(partials, q):
    q8 = partials.shape[0]
    blk = 1000

    def red(p_ref, o_ref):
        r = lax.broadcasted_iota(jnp.int32, (128, 8), 0) // 16
        t = lax.broadcasted_iota(jnp.int32, (128, 8), 1)
        mask = (r == t).astype(jnp.float32)
        o_ref[...] = jnp.dot(p_ref[...], mask,
                             preferred_element_type=jnp.float32)

    out = pl.pallas_call(
        red,
        grid=(q8 // blk,),
        in_specs=[pl.BlockSpec((blk, 128), lambda i: (i, 0))],
        out_specs=pl.BlockSpec((blk, 8), lambda i: (i, 0)),
        out_shape=jax.ShapeDtypeStruct((q8, 8), jnp.float32),
    )(partials)
    return out.reshape(q)


def kernel(x, edge_index, edge_weight, edges, W):
    e = edge_index.shape[1]
    q = edges.shape[1]
    epack = jnp.stack(
        [edge_index[0].reshape(e // _CE, _CE),
         edge_index[1].reshape(e // _CE, _CE)], axis=1)
    ew = edge_weight.reshape(e // _CE, _CE)
    qpack = jnp.stack(
        [edges[0].reshape(q // _CQ, _CQ),
         edges[1].reshape(q // _CQ, _CQ)], axis=1)

    part = _segment_sum_partials(x, epack, ew)
    h = _linear(part[0], part[1], W)
    partials = _pair_partial_dots(h, qpack)
    return _lane_reduce(partials, q)
